# baseline (speedup 1.0000x reference)
import functools

import jax
import jax.numpy as jnp
from jax.experimental import pallas as pl
from jax.experimental.pallas import tpu as pltpu


def _interleave_matrix(tl: int, k: int, dtype):
    rows = k * tl
    i_iota = jax.lax.broadcasted_iota(jnp.int32, (rows, rows), 0)
    j_iota = jax.lax.broadcasted_iota(jnp.int32, (rows, rows), 1)
    o = i_iota // tl
    t = i_iota - o * tl
    return (j_iota == k * t + o).astype(dtype)


def _unpool_kernel(x_ref, idx_ref, o_ref, *, k: int, tl: int):
    x = x_ref[...]
    idx = idx_ref[...]
    tr = x.shape[0]
    t_glob = pl.program_id(1) * tl + jax.lax.broadcasted_iota(
        jnp.int32, (tr, tl), 1)
    base = k * t_glob
    parts = []
    rest = x
    for o in range(k - 1):
        p = jnp.where(idx == base + o, x, 0.0)
        parts.append(p)
        rest = rest - p
    parts.append(rest)
    ed = jnp.concatenate(parts, axis=1)
    g2 = _interleave_matrix(tl, k, x.dtype)
    out = jnp.dot(ed, g2, preferred_element_type=jnp.float32,
                  precision=jax.lax.Precision.DEFAULT)
    o_ref[...] = out.astype(o_ref.dtype)


def kernel(x, indices):
    k = 2
    N, C, L = x.shape
    Lout = L * k
    rows = N * C
    x2 = x.reshape(rows, L)
    idx2 = indices.reshape(rows, L).astype(jnp.int32)

    TR = min(rows, 8192)
    TL = 128 if L % 128 == 0 else L
    TN = TL * k
    grid = (rows // TR, L // TL)
    out2 = pl.pallas_call(
        functools.partial(_unpool_kernel, k=k, tl=TL),
        out_shape=jax.ShapeDtypeStruct((rows, Lout), x.dtype),
        grid=grid,
        in_specs=[
            pl.BlockSpec((TR, TL), lambda r, l: (r, l)),
            pl.BlockSpec((TR, TL), lambda r, l: (r, l)),
        ],
        out_specs=pl.BlockSpec((TR, TN), lambda r, l: (r, l)),
        compiler_params=pltpu.CompilerParams(
            dimension_semantics=("parallel", "parallel"),
            vmem_limit_bytes=100 * 1024 * 1024),
    )(x2, idx2)
    return out2.reshape(N, C, Lout)


# speedup vs baseline: 1.0056x; 1.0056x over previous
import functools

import jax
import jax.numpy as jnp
from jax.experimental import pallas as pl
from jax.experimental.pallas import tpu as pltpu


def _interleave_matrix(tl: int, k: int, dtype):
    rows = k * tl
    i_iota = jax.lax.broadcasted_iota(jnp.int32, (rows, rows), 0)
    j_iota = jax.lax.broadcasted_iota(jnp.int32, (rows, rows), 1)
    o = i_iota // tl
    t = i_iota - o * tl
    return (j_iota == k * t + o).astype(dtype)


def _unpool_kernel(x_ref, idx_ref, o_ref, *, k: int, cl: int):
    tr, l = x_ref.shape
    g2 = _interleave_matrix(cl, k, x_ref.dtype)
    for c in range(l // cl):
        x = x_ref[:, c * cl:(c + 1) * cl]
        idx = idx_ref[:, c * cl:(c + 1) * cl]
        base = k * (c * cl + jax.lax.broadcasted_iota(
            jnp.int32, (tr, cl), 1))
        parts = []
        rest = x
        for o in range(k - 1):
            p = jnp.where(idx == base + o, x, 0.0)
            parts.append(p)
            rest = rest - p
        parts.append(rest)
        ed = jnp.concatenate(parts, axis=1)
        out = jnp.dot(ed, g2, preferred_element_type=jnp.float32,
                      precision=jax.lax.Precision.DEFAULT)
        o_ref[:, c * k * cl:(c + 1) * k * cl] = out.astype(o_ref.dtype)


def kernel(x, indices):
    k = 2
    N, C, L = x.shape
    Lout = L * k
    rows = N * C
    x2 = x.reshape(rows, L)
    idx2 = indices.reshape(rows, L).astype(jnp.int32)

    TR = min(rows, 2048)
    CL = 128 if L % 128 == 0 else L
    grid = (rows // TR,)
    out2 = pl.pallas_call(
        functools.partial(_unpool_kernel, k=k, cl=CL),
        out_shape=jax.ShapeDtypeStruct((rows, Lout), x.dtype),
        grid=grid,
        in_specs=[
            pl.BlockSpec((TR, L), lambda r: (r, 0)),
            pl.BlockSpec((TR, L), lambda r: (r, 0)),
        ],
        out_specs=pl.BlockSpec((TR, Lout), lambda r: (r, 0)),
        compiler_params=pltpu.CompilerParams(
            dimension_semantics=("parallel",),
            vmem_limit_bytes=100 * 1024 * 1024),
    )(x2, idx2)
    return out2.reshape(N, C, Lout)
